# Initial kernel scaffold; baseline (speedup 1.0000x reference)
import jax
import jax.numpy as jnp
from jax.experimental import pallas as pl


def kernel(query_input, padding_mask, W_qk, W_v, W_o, training):
    raise NotImplementedError("write your pallas kernel here")


# speedup vs baseline: 1.1273x; 1.1273x over previous
import functools

import jax
import jax.numpy as jnp
from jax.experimental import pallas as pl

HIDDEN = 1024
NUM_HEADS = 16
DIM_PER_HEAD = HIDDEN // NUM_HEADS
QCHUNK = 512


def _fused_attn_kernel(x_ref, pm_ref, wqk_ref, wv_ref, wo_ref, out_ref):
    n = pl.program_id(1)
    x = x_ref[0]
    wqk = wqk_ref[0]
    wv = wv_ref[0]
    wo = wo_ref[0]
    pm_bias = pm_ref[0]

    L = x.shape[0]
    scale = DIM_PER_HEAD ** -0.5

    q = jnp.dot(x, wqk, preferred_element_type=jnp.float32)
    v = jnp.dot(x, wv, preferred_element_type=jnp.float32)
    norm = jnp.sqrt(jnp.sum(q * q, axis=1, keepdims=True))
    kn = q * (1.0 / jnp.maximum(norm, 1e-12))
    qs = q * scale

    for c in range(L // QCHUNK):
        row0 = c * QCHUNK
        qc = qs[row0:row0 + QCHUNK, :]
        logits = jax.lax.dot_general(
            qc, kn, (((1,), (1,)), ((), ())),
            preferred_element_type=jnp.float32)
        rows = jax.lax.broadcasted_iota(jnp.int32, (QCHUNK, L), 0) + row0
        cols = jax.lax.broadcasted_iota(jnp.int32, (QCHUNK, L), 1)
        logits = logits + jnp.where(rows == cols, -1e5, 0.0) + pm_bias
        m = jnp.max(logits, axis=1, keepdims=True)
        e = jnp.exp(logits - m)
        w = e * (1.0 / jnp.sum(e, axis=1, keepdims=True))
        attn = jnp.dot(w, v, preferred_element_type=jnp.float32)
        contrib = jnp.dot(attn, wo, preferred_element_type=jnp.float32)

        @pl.when(n == 0)
        def _():
            out_ref[0, row0:row0 + QCHUNK, :] = contrib

        @pl.when(n > 0)
        def _():
            out_ref[0, row0:row0 + QCHUNK, :] = (
                out_ref[0, row0:row0 + QCHUNK, :] + contrib)


@functools.partial(jax.jit, static_argnames=("interpret",))
def _run(query_input, pm_bias, wqk_t, wv_t, W_o, interpret=False):
    B, L, D = query_input.shape
    grid = (B, NUM_HEADS)
    return pl.pallas_call(
        _fused_attn_kernel,
        grid=grid,
        in_specs=[
            pl.BlockSpec((1, L, D), lambda b, n: (b, 0, 0)),
            pl.BlockSpec((1, 1, L), lambda b, n: (b, 0, 0)),
            pl.BlockSpec((1, D, DIM_PER_HEAD), lambda b, n: (n, 0, 0)),
            pl.BlockSpec((1, D, DIM_PER_HEAD), lambda b, n: (n, 0, 0)),
            pl.BlockSpec((1, DIM_PER_HEAD, D), lambda b, n: (n, 0, 0)),
        ],
        out_specs=pl.BlockSpec((1, L, D), lambda b, n: (b, 0, 0)),
        out_shape=jax.ShapeDtypeStruct((B, L, D), jnp.float32),
        interpret=interpret,
    )(query_input, pm_bias, wqk_t, wv_t, W_o)


def kernel(query_input, padding_mask, W_qk, W_v, W_o, training=0):
    B, L, _ = query_input.shape
    pm_bias = (padding_mask.astype(jnp.float32) * -1e9).reshape(B, 1, L)
    wqk_t = jnp.transpose(W_qk, (1, 0, 2))
    wv_t = jnp.transpose(W_v, (1, 0, 2))
    return _run(query_input, pm_bias, wqk_t, wv_t, W_o)
